# Initial kernel scaffold; baseline (speedup 1.0000x reference)
import jax
import jax.numpy as jnp
from jax.experimental import pallas as pl


def kernel(hidden_states, W1, b1, W2, b2):
    raise NotImplementedError("write your pallas kernel here")


# speedup vs baseline: 1.6198x; 1.6198x over previous
import functools

import jax
import jax.numpy as jnp
from jax import lax
from jax.experimental import pallas as pl
from jax.experimental.pallas import tpu as pltpu
from jax.experimental.pallas import tpu_sc as plsc

_B = 4
_S = 4096
_HIDDEN = 2048
_H4 = _HIDDEN // 4
_K = max(1, min(int(0.5 * _S), _S))
_TILE_S = 512
_NS = _S // _TILE_S
_L = 16
_NCHUNK = _S // _L

_USE_SC_SELECT = True


def _logits_body(x_ref, w1_ref, b1_ref, w2t_ref, b2_ref, lg_ref):
    x = x_ref[0]
    h = jnp.dot(x, w1_ref[...], preferred_element_type=jnp.float32)
    h = h + b1_ref[...]
    h = 0.5 * h * (1.0 + lax.erf(h * 0.7071067811865476))
    lg8 = lax.dot_general(w2t_ref[...], h, (((1,), (1,)), ((), ())),
                          preferred_element_type=jnp.float32)
    lg_ref[...] = (lg8[0:1] + b2_ref[...])[None]


def _tc_select_body(lg_ref, w_ref, m_ref):
    logits = lg_ref[...]
    bits = lax.bitcast_convert_type(logits, jnp.int32)
    keys = jnp.where(bits >= 0, bits, bits ^ jnp.int32(0x7FFFFFFF))
    idx = lax.broadcasted_iota(jnp.int32, (1, 1, _S), 2)

    def count_ge(t):
        return jnp.sum((keys >= t).astype(jnp.int32))

    def val_step(_, carry):
        lo, hi = carry
        mid = (lo >> 1) + (hi >> 1) + (lo & hi & 1) + ((lo ^ hi) & 1)
        ge = count_ge(mid) >= _K
        return jnp.where(ge, mid, lo), jnp.where(ge, hi, mid - 1)

    tv, _ = lax.fori_loop(
        0, 32, val_step,
        (jnp.int32(-(2 ** 31)), jnp.int32(2 ** 31 - 1)))

    eq = keys == tv
    c_gt = jnp.sum((keys > tv).astype(jnp.int32))
    r = _K - c_gt

    def idx_step(_, carry):
        lo, hi = carry
        mid = (lo + hi) >> 1
        ge = jnp.sum((eq & (idx <= mid)).astype(jnp.int32)) >= r
        return jnp.where(ge, lo, mid + 1), jnp.where(ge, mid, hi)

    _, j = lax.fori_loop(0, 13, idx_step, (jnp.int32(0), jnp.int32(_S - 1)))

    mask = (keys > tv) | (eq & (idx <= j))
    scores = jax.nn.sigmoid(logits)
    w_ref[...] = jnp.where(mask, scores, jnp.float32(0.0))
    m_ref[...] = mask.astype(jnp.int32)


def _tc_select(logits3):
    weights, mask = pl.pallas_call(
        _tc_select_body,
        grid=(_B,),
        in_specs=[pl.BlockSpec((1, 1, _S), lambda b: (b, 0, 0))],
        out_specs=[
            pl.BlockSpec((1, 1, _S), lambda b: (b, 0, 0)),
            pl.BlockSpec((1, 1, _S), lambda b: (b, 0, 0)),
        ],
        out_shape=[
            jax.ShapeDtypeStruct((_B, 1, _S), jnp.float32),
            jax.ShapeDtypeStruct((_B, 1, _S), jnp.int32),
        ],
    )(logits3)
    return weights.reshape(_B, _S), mask.reshape(_B, _S)


def _sc_select_body(lg_hbm, w_hbm, m_hbm, lg_v, keys_v, w_v, m_v):
    wid = lax.axis_index("s") * 2 + lax.axis_index("c")

    @pl.when(wid < _B)
    def _():
        pltpu.sync_copy(lg_hbm.at[wid], lg_v)

        def mk(i, carry):
            v = lg_v[pl.ds(i * _L, _L)]
            b = lax.bitcast_convert_type(v, jnp.int32)
            keys_v[pl.ds(i * _L, _L)] = jnp.where(
                b >= 0, b, b ^ jnp.int32(0x7FFFFFFF))
            return carry

        lax.fori_loop(0, _NCHUNK, mk, jnp.int32(0))

        def count_where(pred):
            def chunk(i, acc):
                k = keys_v[pl.ds(i * _L, _L)]
                gidx = i * _L + lax.iota(jnp.int32, _L)
                return acc + plsc.all_reduce_population_count(pred(k, gidx))
            return lax.fori_loop(0, _NCHUNK, chunk,
                                 jnp.zeros((_L,), jnp.int32))

        def splat(v):
            return jnp.full((_L,), v, jnp.int32)

        kk = splat(_K)

        def val_step(_, carry):
            lo, hi = carry
            mid = (lo >> 1) + (hi >> 1) + (lo & hi & 1) + ((lo ^ hi) & 1)
            ge = count_where(lambda k, g: k >= mid) >= kk
            return jnp.where(ge, mid, lo), jnp.where(ge, hi, mid - 1)

        tv, _ = lax.fori_loop(
            0, 32, val_step,
            (splat(-(2 ** 31)), splat(2 ** 31 - 1)))

        c_gt = count_where(lambda k, g: k > tv)
        r = kk - c_gt

        def idx_step(_, carry):
            lo, hi = carry
            mid = (lo + hi) >> 1
            ge = count_where(lambda k, g: (k == tv) & (g <= mid)) >= r
            return jnp.where(ge, lo, mid + 1), jnp.where(ge, mid, hi)

        _, j = lax.fori_loop(0, 13, idx_step,
                             (splat(0), splat(_S - 1)))

        def emit(i, carry):
            sl = pl.ds(i * _L, _L)
            k = keys_v[sl]
            gidx = i * _L + lax.iota(jnp.int32, _L)
            mask = (k > tv) | ((k == tv) & (gidx <= j))
            x = lg_v[sl]
            score = 1.0 / (1.0 + jnp.exp(-x))
            w_v[sl] = jnp.where(mask, score, jnp.float32(0.0))
            m_v[sl] = mask.astype(jnp.int32)
            return carry

        lax.fori_loop(0, _NCHUNK, emit, jnp.int32(0))
        pltpu.sync_copy(w_v, w_hbm.at[wid])
        pltpu.sync_copy(m_v, m_hbm.at[wid])


def _sc_select(logits3):
    logits2 = logits3.reshape(_B, _S)
    sel = pl.kernel(
        _sc_select_body,
        out_type=[
            jax.ShapeDtypeStruct((_B, _S), jnp.float32),
            jax.ShapeDtypeStruct((_B, _S), jnp.int32),
        ],
        mesh=plsc.VectorSubcoreMesh(core_axis_name="c", subcore_axis_name="s"),
        compiler_params=pltpu.CompilerParams(needs_layout_passes=False),
        scratch_types=[
            pltpu.VMEM((_S,), jnp.float32),
            pltpu.VMEM((_S,), jnp.int32),
            pltpu.VMEM((_S,), jnp.float32),
            pltpu.VMEM((_S,), jnp.int32),
        ],
    )
    return sel(logits2)


def kernel(hidden_states, W1, b1, W2, b2):
    b1r = b1.reshape(1, _H4)
    w2t = jnp.zeros((8, _H4), jnp.float32).at[0].set(W2[:, 0])
    b2r = b2.reshape(1, 1)

    logits3 = pl.pallas_call(
        _logits_body,
        grid=(_B, _NS),
        in_specs=[
            pl.BlockSpec((1, _TILE_S, _HIDDEN), lambda b, s: (b, s, 0)),
            pl.BlockSpec((_HIDDEN, _H4), lambda b, s: (0, 0)),
            pl.BlockSpec((1, _H4), lambda b, s: (0, 0)),
            pl.BlockSpec((8, _H4), lambda b, s: (0, 0)),
            pl.BlockSpec((1, 1), lambda b, s: (0, 0)),
        ],
        out_specs=pl.BlockSpec((1, 1, _TILE_S), lambda b, s: (b, 0, s)),
        out_shape=jax.ShapeDtypeStruct((_B, 1, _S), jnp.float32),
    )(hidden_states, W1, b1r, w2t, b2r)

    if _USE_SC_SELECT:
        weights, mask = _sc_select(logits3)
    else:
        weights, mask = _tc_select(logits3)
    return (weights.reshape(_B, _S), mask.reshape(_B, _S).astype(bool))


# speedup vs baseline: 2.0909x; 1.2908x over previous
import functools

import jax
import jax.numpy as jnp
from jax import lax
from jax.experimental import pallas as pl
from jax.experimental.pallas import tpu as pltpu
from jax.experimental.pallas import tpu_sc as plsc

_B = 4
_S = 4096
_HIDDEN = 2048
_H4 = _HIDDEN // 4
_K = max(1, min(int(0.5 * _S), _S))
_TILE_S = 512
_NS = _S // _TILE_S
_L = 16
_NCHUNK = _S // _L
_UNROLL = 8

_USE_SC_SELECT = "hist"


def _logits_body(x_ref, w1_ref, b1_ref, w2t_ref, b2_ref, lg_ref):
    x = x_ref[0]
    h = jnp.dot(x, w1_ref[...], preferred_element_type=jnp.float32)
    h = h + b1_ref[...]
    h = 0.5 * h * (1.0 + lax.erf(h * 0.7071067811865476))
    lg8 = lax.dot_general(w2t_ref[...], h, (((1,), (1,)), ((), ())),
                          preferred_element_type=jnp.float32)
    lg_ref[...] = (lg8[0:1] + b2_ref[...])[None]


def _tc_select_body(lg_ref, w_ref, m_ref):
    logits = lg_ref[...]
    zc = jnp.where(logits == 0.0, jnp.float32(0.0), logits)
    bits = lax.bitcast_convert_type(zc, jnp.int32)
    keys = jnp.where(bits >= 0, bits, bits ^ jnp.int32(0x7FFFFFFF))
    idx = lax.broadcasted_iota(jnp.int32, (1, 1, _S), 2)

    def count_ge(t):
        return jnp.sum((keys >= t).astype(jnp.int32))

    def val_step(_, carry):
        lo, hi = carry
        mid = (lo >> 1) + (hi >> 1) + (lo & hi & 1) + ((lo ^ hi) & 1)
        ge = count_ge(mid) >= _K
        return jnp.where(ge, mid, lo), jnp.where(ge, hi, mid - 1)

    tv, _ = lax.fori_loop(
        0, 32, val_step,
        (jnp.int32(-(2 ** 31)), jnp.int32(2 ** 31 - 1)))

    eq = keys == tv
    c_gt = jnp.sum((keys > tv).astype(jnp.int32))
    r = _K - c_gt

    def idx_step(_, carry):
        lo, hi = carry
        mid = (lo + hi) >> 1
        ge = jnp.sum((eq & (idx <= mid)).astype(jnp.int32)) >= r
        return jnp.where(ge, lo, mid + 1), jnp.where(ge, mid, hi)

    _, j = lax.fori_loop(0, 13, idx_step, (jnp.int32(0), jnp.int32(_S - 1)))

    mask = (keys > tv) | (eq & (idx <= j))
    scores = jax.nn.sigmoid(logits)
    w_ref[...] = jnp.where(mask, scores, jnp.float32(0.0))
    m_ref[...] = mask.astype(jnp.int32)


def _tc_select(logits3):
    weights, mask = pl.pallas_call(
        _tc_select_body,
        grid=(_B,),
        in_specs=[pl.BlockSpec((1, 1, _S), lambda b: (b, 0, 0))],
        out_specs=[
            pl.BlockSpec((1, 1, _S), lambda b: (b, 0, 0)),
            pl.BlockSpec((1, 1, _S), lambda b: (b, 0, 0)),
        ],
        out_shape=[
            jax.ShapeDtypeStruct((_B, 1, _S), jnp.float32),
            jax.ShapeDtypeStruct((_B, 1, _S), jnp.int32),
        ],
    )(logits3)
    return weights.reshape(_B, _S), mask.reshape(_B, _S)


def _sc_select_body(lg_hbm, w_hbm, m_hbm, lg_v, keys_v, w_v, m_v):
    wid = lax.axis_index("s") * 2 + lax.axis_index("c")

    @pl.when(wid < _B)
    def _():
        pltpu.sync_copy(lg_hbm.at[wid], lg_v)

        def mk(i, carry):
            for u in range(_UNROLL):
                sl = pl.ds((i * _UNROLL + u) * _L, _L)
                v = lg_v[sl]
                v = jnp.where(v == 0.0, jnp.float32(0.0), v)
                b = lax.bitcast_convert_type(v, jnp.int32)
                keys_v[sl] = jnp.where(b >= 0, b, b ^ jnp.int32(0x7FFFFFFF))
            return carry

        lax.fori_loop(0, _NCHUNK // _UNROLL, mk, jnp.int32(0))

        def count_where(pred):
            def chunk(i, acc):
                for u in range(_UNROLL):
                    c = i * _UNROLL + u
                    k = keys_v[pl.ds(c * _L, _L)]
                    gidx = c * _L + lax.iota(jnp.int32, _L)
                    acc = acc + plsc.all_reduce_population_count(pred(k, gidx))
                return acc
            return lax.fori_loop(0, _NCHUNK // _UNROLL, chunk,
                                 jnp.zeros((_L,), jnp.int32))

        def splat(v):
            return jnp.full((_L,), v, jnp.int32)

        kk = splat(_K)

        def val_step(_, carry):
            lo, hi = carry
            mid = (lo >> 1) + (hi >> 1) + (lo & hi & 1) + ((lo ^ hi) & 1)
            ge = count_where(lambda k, g: k >= mid) >= kk
            return jnp.where(ge, mid, lo), jnp.where(ge, hi, mid - 1)

        tv, _ = lax.fori_loop(
            0, 32, val_step,
            (splat(-(2 ** 31)), splat(2 ** 31 - 1)))

        c_gt = count_where(lambda k, g: k > tv)
        r = kk - c_gt

        def idx_step(_, carry):
            lo, hi = carry
            mid = (lo + hi) >> 1
            ge = count_where(lambda k, g: (k == tv) & (g <= mid)) >= r
            return jnp.where(ge, lo, mid + 1), jnp.where(ge, mid, hi)

        _, j = lax.fori_loop(0, 13, idx_step,
                             (splat(0), splat(_S - 1)))

        def emit(i, carry):
            for u in range(_UNROLL):
                c = i * _UNROLL + u
                sl = pl.ds(c * _L, _L)
                k = keys_v[sl]
                gidx = c * _L + lax.iota(jnp.int32, _L)
                mask = (k > tv) | ((k == tv) & (gidx <= j))
                x = lg_v[sl]
                score = 1.0 / (1.0 + jnp.exp(-x))
                w_v[sl] = jnp.where(mask, score, jnp.float32(0.0))
                m_v[sl] = mask.astype(jnp.int32)
            return carry

        lax.fori_loop(0, _NCHUNK // _UNROLL, emit, jnp.int32(0))
        pltpu.sync_copy(w_v, w_hbm.at[wid])
        pltpu.sync_copy(m_v, m_hbm.at[wid])


def _sc_hist_body(lg_hbm, w_hbm, m_hbm, lg_v, keys_v, hist_v, ck_v, ci_v,
                  w_v, m_v):
    wid = lax.axis_index("s") * 2 + lax.axis_index("c")

    @pl.when(wid < _B)
    def _():
        pltpu.sync_copy(lg_hbm.at[wid], lg_v)

        zero16 = jnp.zeros((_L,), jnp.int32)
        one16 = jnp.full((_L,), 1, jnp.int32)
        lane = lax.iota(jnp.int32, _L)

        def clr(i, carry):
            for u in range(_UNROLL):
                hist_v[pl.ds((i * _UNROLL + u) * _L, _L)] = zero16
            return carry

        lax.fori_loop(0, 256 // _UNROLL, clr, jnp.int32(0))

        def mk(i, carry):
            for u in range(_UNROLL):
                sl = pl.ds((i * _UNROLL + u) * _L, _L)
                v = lg_v[sl]
                v = jnp.where(v == 0.0, jnp.float32(0.0), v)
                b = lax.bitcast_convert_type(v, jnp.int32)
                k = jnp.where(b >= 0, b, b ^ jnp.int32(0x7FFFFFFF))
                keys_v[sl] = k
                addr = (((k >> 24) + 128) << 4) + lane
                plsc.addupdate_scatter(hist_v, [addr], one16)
            return carry

        lax.fori_loop(0, _NCHUNK // _UNROLL, mk, jnp.int32(0))

        def bscan(i, carry):
            csum, tb, c_above, nc = carry
            b = 255 - i
            h = hist_v[pl.ds(b * _L, _L)]
            s = jnp.sum(h)
            upd = (tb < 0) & (csum + s >= _K)
            tb = jnp.where(upd, b, tb)
            c_above = jnp.where(upd, csum, c_above)
            nc = jnp.where(upd, s, nc)
            csum = csum + jnp.where(tb < 0, s, 0)
            return csum, tb, c_above, nc

        _, tb, c_above, nc = lax.fori_loop(
            0, 256, bscan,
            (jnp.int32(0), jnp.int32(-1), jnp.int32(0), jnp.int32(0)))

        def compact(i, off):
            for u in range(_UNROLL):
                c = i * _UNROLL + u
                sl = pl.ds(c * _L, _L)
                k = keys_v[sl]
                mm = ((k >> 24) + 128) == tb
                plsc.store_compressed(ck_v.at[pl.ds(off, _L)], k, mask=mm)
                plsc.store_compressed(ci_v.at[pl.ds(off, _L)], c * _L + lane,
                                      mask=mm)
                off = off + jnp.sum(mm.astype(jnp.int32))
            return off

        lax.fori_loop(0, _NCHUNK // _UNROLL, compact, jnp.int32(0))

        ncc = (nc + _L - 1) >> 4

        def count_cand(pred):
            def chunk(i, acc):
                sl = pl.ds(i * _L, _L)
                k = ck_v[sl]
                gi = ci_v[sl]
                valid = (i * _L + lane) < nc
                return acc + plsc.all_reduce_population_count(
                    pred(k, gi) & valid)
            return lax.fori_loop(0, ncc, chunk, zero16)

        def splat(v):
            return jnp.full((_L,), v, jnp.int32)

        r1 = splat(_K - c_above)

        def val_step(_, carry):
            lo, hi = carry
            mid = (lo >> 1) + (hi >> 1) + (lo & hi & 1) + ((lo ^ hi) & 1)
            ge = count_cand(lambda k, g: k >= mid) >= r1
            return jnp.where(ge, mid, lo), jnp.where(ge, hi, mid - 1)

        tv, _ = lax.fori_loop(
            0, 32, val_step,
            (splat(-(2 ** 31)), splat(2 ** 31 - 1)))

        c_gt = splat(c_above) + count_cand(lambda k, g: k > tv)
        r = splat(_K) - c_gt

        def idx_step(_, carry):
            lo, hi = carry
            mid = (lo + hi) >> 1
            ge = count_cand(lambda k, g: (k == tv) & (g <= mid)) >= r
            return jnp.where(ge, lo, mid + 1), jnp.where(ge, mid, hi)

        _, j = lax.fori_loop(0, 13, idx_step, (splat(0), splat(_S - 1)))

        def emit(i, carry):
            for u in range(_UNROLL):
                c = i * _UNROLL + u
                sl = pl.ds(c * _L, _L)
                k = keys_v[sl]
                gidx = c * _L + lane
                mask = (k > tv) | ((k == tv) & (gidx <= j))
                x = lg_v[sl]
                score = 1.0 / (1.0 + jnp.exp(-x))
                w_v[sl] = jnp.where(mask, score, jnp.float32(0.0))
                m_v[sl] = mask.astype(jnp.int32)
            return carry

        lax.fori_loop(0, _NCHUNK // _UNROLL, emit, jnp.int32(0))
        pltpu.sync_copy(w_v, w_hbm.at[wid])
        pltpu.sync_copy(m_v, m_hbm.at[wid])


def _sc_select_hist(logits3):
    logits2 = logits3.reshape(_B, _S)
    sel = pl.kernel(
        _sc_hist_body,
        out_type=[
            jax.ShapeDtypeStruct((_B, _S), jnp.float32),
            jax.ShapeDtypeStruct((_B, _S), jnp.int32),
        ],
        mesh=plsc.VectorSubcoreMesh(core_axis_name="c", subcore_axis_name="s"),
        compiler_params=pltpu.CompilerParams(needs_layout_passes=False),
        scratch_types=[
            pltpu.VMEM((_S,), jnp.float32),
            pltpu.VMEM((_S,), jnp.int32),
            pltpu.VMEM((256 * _L,), jnp.int32),
            pltpu.VMEM((_S + _L,), jnp.int32),
            pltpu.VMEM((_S + _L,), jnp.int32),
            pltpu.VMEM((_S,), jnp.float32),
            pltpu.VMEM((_S,), jnp.int32),
        ],
    )
    return sel(logits2)


def _sc_select(logits3):
    logits2 = logits3.reshape(_B, _S)
    sel = pl.kernel(
        _sc_select_body,
        out_type=[
            jax.ShapeDtypeStruct((_B, _S), jnp.float32),
            jax.ShapeDtypeStruct((_B, _S), jnp.int32),
        ],
        mesh=plsc.VectorSubcoreMesh(core_axis_name="c", subcore_axis_name="s"),
        compiler_params=pltpu.CompilerParams(needs_layout_passes=False),
        scratch_types=[
            pltpu.VMEM((_S,), jnp.float32),
            pltpu.VMEM((_S,), jnp.int32),
            pltpu.VMEM((_S,), jnp.float32),
            pltpu.VMEM((_S,), jnp.int32),
        ],
    )
    return sel(logits2)


def kernel(hidden_states, W1, b1, W2, b2):
    b1r = b1.reshape(1, _H4)
    w2t = jnp.zeros((8, _H4), jnp.float32).at[0].set(W2[:, 0])
    b2r = b2.reshape(1, 1)

    logits3 = pl.pallas_call(
        _logits_body,
        grid=(_B, _NS),
        in_specs=[
            pl.BlockSpec((1, _TILE_S, _HIDDEN), lambda b, s: (b, s, 0)),
            pl.BlockSpec((_HIDDEN, _H4), lambda b, s: (0, 0)),
            pl.BlockSpec((1, _H4), lambda b, s: (0, 0)),
            pl.BlockSpec((8, _H4), lambda b, s: (0, 0)),
            pl.BlockSpec((1, 1), lambda b, s: (0, 0)),
        ],
        out_specs=pl.BlockSpec((1, 1, _TILE_S), lambda b, s: (b, 0, s)),
        out_shape=jax.ShapeDtypeStruct((_B, 1, _S), jnp.float32),
    )(hidden_states, W1, b1r, w2t, b2r)

    if _USE_SC_SELECT == "hist":
        weights, mask = _sc_select_hist(logits3)
    elif _USE_SC_SELECT == "bisect":
        weights, mask = _sc_select(logits3)
    else:
        weights, mask = _tc_select(logits3)
    return (weights.reshape(_B, _S), mask.reshape(_B, _S).astype(bool))


# speedup vs baseline: 2.3043x; 1.1021x over previous
import functools

import jax
import jax.numpy as jnp
from jax import lax
from jax.experimental import pallas as pl
from jax.experimental.pallas import tpu as pltpu
from jax.experimental.pallas import tpu_sc as plsc

_B = 4
_S = 4096
_HIDDEN = 2048
_H4 = _HIDDEN // 4
_K = max(1, min(int(0.5 * _S), _S))
_TILE_S = 1024
_NS = _S // _TILE_S
_L = 16
_NCHUNK = _S // _L
_UNROLL = 8

_USE_SC_SELECT = "hist"


def _logits_body(x_ref, w1_ref, b1_ref, w2t_ref, b2_ref, lg_ref):
    x = x_ref[0]
    h = jnp.dot(x, w1_ref[...], preferred_element_type=jnp.float32)
    h = h + b1_ref[...]
    h = 0.5 * h * (1.0 + lax.erf(h * 0.7071067811865476))
    lg8 = lax.dot_general(w2t_ref[...], h, (((1,), (1,)), ((), ())),
                          preferred_element_type=jnp.float32)
    lg_ref[...] = (lg8[0:1] + b2_ref[...])[None]


def _tc_select_body(lg_ref, w_ref, m_ref):
    logits = lg_ref[...]
    zc = jnp.where(logits == 0.0, jnp.float32(0.0), logits)
    bits = lax.bitcast_convert_type(zc, jnp.int32)
    keys = jnp.where(bits >= 0, bits, bits ^ jnp.int32(0x7FFFFFFF))
    idx = lax.broadcasted_iota(jnp.int32, (1, 1, _S), 2)

    def count_ge(t):
        return jnp.sum((keys >= t).astype(jnp.int32))

    def val_step(_, carry):
        lo, hi = carry
        mid = (lo >> 1) + (hi >> 1) + (lo & hi & 1) + ((lo ^ hi) & 1)
        ge = count_ge(mid) >= _K
        return jnp.where(ge, mid, lo), jnp.where(ge, hi, mid - 1)

    tv, _ = lax.fori_loop(
        0, 32, val_step,
        (jnp.int32(-(2 ** 31)), jnp.int32(2 ** 31 - 1)))

    eq = keys == tv
    c_gt = jnp.sum((keys > tv).astype(jnp.int32))
    r = _K - c_gt

    def idx_step(_, carry):
        lo, hi = carry
        mid = (lo + hi) >> 1
        ge = jnp.sum((eq & (idx <= mid)).astype(jnp.int32)) >= r
        return jnp.where(ge, lo, mid + 1), jnp.where(ge, mid, hi)

    _, j = lax.fori_loop(0, 13, idx_step, (jnp.int32(0), jnp.int32(_S - 1)))

    mask = (keys > tv) | (eq & (idx <= j))
    scores = jax.nn.sigmoid(logits)
    w_ref[...] = jnp.where(mask, scores, jnp.float32(0.0))
    m_ref[...] = mask.astype(jnp.int32)


def _tc_select(logits3):
    weights, mask = pl.pallas_call(
        _tc_select_body,
        grid=(_B,),
        in_specs=[pl.BlockSpec((1, 1, _S), lambda b: (b, 0, 0))],
        out_specs=[
            pl.BlockSpec((1, 1, _S), lambda b: (b, 0, 0)),
            pl.BlockSpec((1, 1, _S), lambda b: (b, 0, 0)),
        ],
        out_shape=[
            jax.ShapeDtypeStruct((_B, 1, _S), jnp.float32),
            jax.ShapeDtypeStruct((_B, 1, _S), jnp.int32),
        ],
    )(logits3)
    return weights.reshape(_B, _S), mask.reshape(_B, _S)


def _sc_select_body(lg_hbm, w_hbm, m_hbm, lg_v, keys_v, w_v, m_v):
    wid = lax.axis_index("s") * 2 + lax.axis_index("c")

    @pl.when(wid < _B)
    def _():
        pltpu.sync_copy(lg_hbm.at[wid], lg_v)

        def mk(i, carry):
            for u in range(_UNROLL):
                sl = pl.ds((i * _UNROLL + u) * _L, _L)
                v = lg_v[sl]
                v = jnp.where(v == 0.0, jnp.float32(0.0), v)
                b = lax.bitcast_convert_type(v, jnp.int32)
                keys_v[sl] = jnp.where(b >= 0, b, b ^ jnp.int32(0x7FFFFFFF))
            return carry

        lax.fori_loop(0, _NCHUNK // _UNROLL, mk, jnp.int32(0))

        def count_where(pred):
            def chunk(i, acc):
                for u in range(_UNROLL):
                    c = i * _UNROLL + u
                    k = keys_v[pl.ds(c * _L, _L)]
                    gidx = c * _L + lax.iota(jnp.int32, _L)
                    acc = acc + plsc.all_reduce_population_count(pred(k, gidx))
                return acc
            return lax.fori_loop(0, _NCHUNK // _UNROLL, chunk,
                                 jnp.zeros((_L,), jnp.int32))

        def splat(v):
            return jnp.full((_L,), v, jnp.int32)

        kk = splat(_K)

        def val_step(_, carry):
            lo, hi = carry
            mid = (lo >> 1) + (hi >> 1) + (lo & hi & 1) + ((lo ^ hi) & 1)
            ge = count_where(lambda k, g: k >= mid) >= kk
            return jnp.where(ge, mid, lo), jnp.where(ge, hi, mid - 1)

        tv, _ = lax.fori_loop(
            0, 32, val_step,
            (splat(-(2 ** 31)), splat(2 ** 31 - 1)))

        c_gt = count_where(lambda k, g: k > tv)
        r = kk - c_gt

        def idx_step(_, carry):
            lo, hi = carry
            mid = (lo + hi) >> 1
            ge = count_where(lambda k, g: (k == tv) & (g <= mid)) >= r
            return jnp.where(ge, lo, mid + 1), jnp.where(ge, mid, hi)

        _, j = lax.fori_loop(0, 13, idx_step,
                             (splat(0), splat(_S - 1)))

        def emit(i, carry):
            for u in range(_UNROLL):
                c = i * _UNROLL + u
                sl = pl.ds(c * _L, _L)
                k = keys_v[sl]
                gidx = c * _L + lax.iota(jnp.int32, _L)
                mask = (k > tv) | ((k == tv) & (gidx <= j))
                x = lg_v[sl]
                score = 1.0 / (1.0 + jnp.exp(-x))
                w_v[sl] = jnp.where(mask, score, jnp.float32(0.0))
                m_v[sl] = mask.astype(jnp.int32)
            return carry

        lax.fori_loop(0, _NCHUNK // _UNROLL, emit, jnp.int32(0))
        pltpu.sync_copy(w_v, w_hbm.at[wid])
        pltpu.sync_copy(m_v, m_hbm.at[wid])


def _sc_hist_body(lg_hbm, w_hbm, m_hbm, lg_v, keys_v, hist_v, ck_v, ci_v,
                  w_v, m_v):
    wid = lax.axis_index("s") * 2 + lax.axis_index("c")

    @pl.when(wid < _B)
    def _():
        pltpu.sync_copy(lg_hbm.at[wid], lg_v)

        zero16 = jnp.zeros((_L,), jnp.int32)
        one16 = jnp.full((_L,), 1, jnp.int32)
        lane = lax.iota(jnp.int32, _L)

        def clr(i, carry):
            for u in range(_UNROLL):
                hist_v[pl.ds((i * _UNROLL + u) * _L, _L)] = zero16
            return carry

        lax.fori_loop(0, 256 // _UNROLL, clr, jnp.int32(0))

        def mk(i, carry):
            for u in range(_UNROLL):
                sl = pl.ds((i * _UNROLL + u) * _L, _L)
                v = lg_v[sl]
                v = jnp.where(v == 0.0, jnp.float32(0.0), v)
                b = lax.bitcast_convert_type(v, jnp.int32)
                k = jnp.where(b >= 0, b, b ^ jnp.int32(0x7FFFFFFF))
                keys_v[sl] = k
                addr = (((k >> 24) + 128) << 4) + lane
                plsc.addupdate_scatter(hist_v, [addr], one16)
            return carry

        lax.fori_loop(0, _NCHUNK // _UNROLL, mk, jnp.int32(0))

        def bscan(i, carry):
            csum, tb, c_above, nc = carry
            b = 255 - i
            h = hist_v[pl.ds(b * _L, _L)]
            s = jnp.sum(h)
            upd = (tb < 0) & (csum + s >= _K)
            tb = jnp.where(upd, b, tb)
            c_above = jnp.where(upd, csum, c_above)
            nc = jnp.where(upd, s, nc)
            csum = csum + jnp.where(tb < 0, s, 0)
            return csum, tb, c_above, nc

        _, tb, c_above, nc = lax.fori_loop(
            0, 256, bscan,
            (jnp.int32(0), jnp.int32(-1), jnp.int32(0), jnp.int32(0)))

        def compact(i, off):
            for u in range(_UNROLL):
                c = i * _UNROLL + u
                sl = pl.ds(c * _L, _L)
                k = keys_v[sl]
                mm = ((k >> 24) + 128) == tb
                plsc.store_compressed(ck_v.at[pl.ds(off, _L)], k, mask=mm)
                plsc.store_compressed(ci_v.at[pl.ds(off, _L)], c * _L + lane,
                                      mask=mm)
                off = off + jnp.sum(mm.astype(jnp.int32))
            return off

        lax.fori_loop(0, _NCHUNK // _UNROLL, compact, jnp.int32(0))

        ncc = (nc + _L - 1) >> 4

        def count_cand(pred):
            def chunk(i, acc):
                sl = pl.ds(i * _L, _L)
                k = ck_v[sl]
                gi = ci_v[sl]
                valid = (i * _L + lane) < nc
                return acc + plsc.all_reduce_population_count(
                    pred(k, gi) & valid)
            return lax.fori_loop(0, ncc, chunk, zero16)

        def splat(v):
            return jnp.full((_L,), v, jnp.int32)

        r1 = splat(_K - c_above)

        def val_step(_, carry):
            lo, hi = carry
            mid = (lo >> 1) + (hi >> 1) + (lo & hi & 1) + ((lo ^ hi) & 1)
            ge = count_cand(lambda k, g: k >= mid) >= r1
            return jnp.where(ge, mid, lo), jnp.where(ge, hi, mid - 1)

        tv, _ = lax.fori_loop(
            0, 32, val_step,
            (splat(-(2 ** 31)), splat(2 ** 31 - 1)))

        c_gt = splat(c_above) + count_cand(lambda k, g: k > tv)
        r = splat(_K) - c_gt

        def idx_step(_, carry):
            lo, hi = carry
            mid = (lo + hi) >> 1
            ge = count_cand(lambda k, g: (k == tv) & (g <= mid)) >= r
            return jnp.where(ge, lo, mid + 1), jnp.where(ge, mid, hi)

        _, j = lax.fori_loop(0, 13, idx_step, (splat(0), splat(_S - 1)))

        def emit(i, carry):
            for u in range(_UNROLL):
                c = i * _UNROLL + u
                sl = pl.ds(c * _L, _L)
                k = keys_v[sl]
                gidx = c * _L + lane
                mask = (k > tv) | ((k == tv) & (gidx <= j))
                x = lg_v[sl]
                score = 1.0 / (1.0 + jnp.exp(-x))
                w_v[sl] = jnp.where(mask, score, jnp.float32(0.0))
                m_v[sl] = mask.astype(jnp.int32)
            return carry

        lax.fori_loop(0, _NCHUNK // _UNROLL, emit, jnp.int32(0))
        pltpu.sync_copy(w_v, w_hbm.at[wid])
        pltpu.sync_copy(m_v, m_hbm.at[wid])


def _sc_select_hist(logits3):
    logits2 = logits3.reshape(_B, _S)
    sel = pl.kernel(
        _sc_hist_body,
        out_type=[
            jax.ShapeDtypeStruct((_B, _S), jnp.float32),
            jax.ShapeDtypeStruct((_B, _S), jnp.int32),
        ],
        mesh=plsc.VectorSubcoreMesh(core_axis_name="c", subcore_axis_name="s"),
        compiler_params=pltpu.CompilerParams(needs_layout_passes=False),
        scratch_types=[
            pltpu.VMEM((_S,), jnp.float32),
            pltpu.VMEM((_S,), jnp.int32),
            pltpu.VMEM((256 * _L,), jnp.int32),
            pltpu.VMEM((_S + _L,), jnp.int32),
            pltpu.VMEM((_S + _L,), jnp.int32),
            pltpu.VMEM((_S,), jnp.float32),
            pltpu.VMEM((_S,), jnp.int32),
        ],
    )
    return sel(logits2)


def _sc_select(logits3):
    logits2 = logits3.reshape(_B, _S)
    sel = pl.kernel(
        _sc_select_body,
        out_type=[
            jax.ShapeDtypeStruct((_B, _S), jnp.float32),
            jax.ShapeDtypeStruct((_B, _S), jnp.int32),
        ],
        mesh=plsc.VectorSubcoreMesh(core_axis_name="c", subcore_axis_name="s"),
        compiler_params=pltpu.CompilerParams(needs_layout_passes=False),
        scratch_types=[
            pltpu.VMEM((_S,), jnp.float32),
            pltpu.VMEM((_S,), jnp.int32),
            pltpu.VMEM((_S,), jnp.float32),
            pltpu.VMEM((_S,), jnp.int32),
        ],
    )
    return sel(logits2)


def kernel(hidden_states, W1, b1, W2, b2):
    b1r = b1.reshape(1, _H4)
    w2t = jnp.zeros((8, _H4), jnp.float32).at[0].set(W2[:, 0])
    b2r = b2.reshape(1, 1)

    logits3 = pl.pallas_call(
        _logits_body,
        grid=(_B, _NS),
        in_specs=[
            pl.BlockSpec((1, _TILE_S, _HIDDEN), lambda b, s: (b, s, 0)),
            pl.BlockSpec((_HIDDEN, _H4), lambda b, s: (0, 0)),
            pl.BlockSpec((1, _H4), lambda b, s: (0, 0)),
            pl.BlockSpec((8, _H4), lambda b, s: (0, 0)),
            pl.BlockSpec((1, 1), lambda b, s: (0, 0)),
        ],
        out_specs=pl.BlockSpec((1, 1, _TILE_S), lambda b, s: (b, 0, s)),
        out_shape=jax.ShapeDtypeStruct((_B, 1, _S), jnp.float32),
    )(hidden_states, W1, b1r, w2t, b2r)

    if _USE_SC_SELECT == "hist":
        weights, mask = _sc_select_hist(logits3)
    elif _USE_SC_SELECT == "bisect":
        weights, mask = _sc_select(logits3)
    else:
        weights, mask = _tc_select(logits3)
    return (weights.reshape(_B, _S), mask.reshape(_B, _S).astype(bool))


# speedup vs baseline: 2.3417x; 1.0162x over previous
import functools

import jax
import jax.numpy as jnp
from jax import lax
from jax.experimental import pallas as pl
from jax.experimental.pallas import tpu as pltpu
from jax.experimental.pallas import tpu_sc as plsc

_B = 4
_S = 4096
_HIDDEN = 2048
_H4 = _HIDDEN // 4
_K = max(1, min(int(0.5 * _S), _S))
_TILE_S = 2048
_NS = _S // _TILE_S
_L = 16
_NCHUNK = _S // _L
_UNROLL = 8

_USE_SC_SELECT = "hist"


def _logits_body(x_ref, w1_ref, b1_ref, w2t_ref, b2_ref, lg_ref):
    x = x_ref[0]
    h = jnp.dot(x, w1_ref[...], preferred_element_type=jnp.float32)
    h = h + b1_ref[...]
    h = 0.5 * h * (1.0 + lax.erf(h * 0.7071067811865476))
    lg8 = lax.dot_general(w2t_ref[...], h, (((1,), (1,)), ((), ())),
                          preferred_element_type=jnp.float32)
    lg_ref[...] = (lg8[0:1] + b2_ref[...])[None]


def _tc_select_body(lg_ref, w_ref, m_ref):
    logits = lg_ref[...]
    zc = jnp.where(logits == 0.0, jnp.float32(0.0), logits)
    bits = lax.bitcast_convert_type(zc, jnp.int32)
    keys = jnp.where(bits >= 0, bits, bits ^ jnp.int32(0x7FFFFFFF))
    idx = lax.broadcasted_iota(jnp.int32, (1, 1, _S), 2)

    def count_ge(t):
        return jnp.sum((keys >= t).astype(jnp.int32))

    def val_step(_, carry):
        lo, hi = carry
        mid = (lo >> 1) + (hi >> 1) + (lo & hi & 1) + ((lo ^ hi) & 1)
        ge = count_ge(mid) >= _K
        return jnp.where(ge, mid, lo), jnp.where(ge, hi, mid - 1)

    tv, _ = lax.fori_loop(
        0, 32, val_step,
        (jnp.int32(-(2 ** 31)), jnp.int32(2 ** 31 - 1)))

    eq = keys == tv
    c_gt = jnp.sum((keys > tv).astype(jnp.int32))
    r = _K - c_gt

    def idx_step(_, carry):
        lo, hi = carry
        mid = (lo + hi) >> 1
        ge = jnp.sum((eq & (idx <= mid)).astype(jnp.int32)) >= r
        return jnp.where(ge, lo, mid + 1), jnp.where(ge, mid, hi)

    _, j = lax.fori_loop(0, 13, idx_step, (jnp.int32(0), jnp.int32(_S - 1)))

    mask = (keys > tv) | (eq & (idx <= j))
    scores = jax.nn.sigmoid(logits)
    w_ref[...] = jnp.where(mask, scores, jnp.float32(0.0))
    m_ref[...] = mask.astype(jnp.int32)


def _tc_select(logits3):
    weights, mask = pl.pallas_call(
        _tc_select_body,
        grid=(_B,),
        in_specs=[pl.BlockSpec((1, 1, _S), lambda b: (b, 0, 0))],
        out_specs=[
            pl.BlockSpec((1, 1, _S), lambda b: (b, 0, 0)),
            pl.BlockSpec((1, 1, _S), lambda b: (b, 0, 0)),
        ],
        out_shape=[
            jax.ShapeDtypeStruct((_B, 1, _S), jnp.float32),
            jax.ShapeDtypeStruct((_B, 1, _S), jnp.int32),
        ],
    )(logits3)
    return weights.reshape(_B, _S), mask.reshape(_B, _S)


def _sc_select_body(lg_hbm, w_hbm, m_hbm, lg_v, keys_v, w_v, m_v):
    wid = lax.axis_index("s") * 2 + lax.axis_index("c")

    @pl.when(wid < _B)
    def _():
        pltpu.sync_copy(lg_hbm.at[wid], lg_v)

        def mk(i, carry):
            for u in range(_UNROLL):
                sl = pl.ds((i * _UNROLL + u) * _L, _L)
                v = lg_v[sl]
                v = jnp.where(v == 0.0, jnp.float32(0.0), v)
                b = lax.bitcast_convert_type(v, jnp.int32)
                keys_v[sl] = jnp.where(b >= 0, b, b ^ jnp.int32(0x7FFFFFFF))
            return carry

        lax.fori_loop(0, _NCHUNK // _UNROLL, mk, jnp.int32(0))

        def count_where(pred):
            def chunk(i, acc):
                for u in range(_UNROLL):
                    c = i * _UNROLL + u
                    k = keys_v[pl.ds(c * _L, _L)]
                    gidx = c * _L + lax.iota(jnp.int32, _L)
                    acc = acc + plsc.all_reduce_population_count(pred(k, gidx))
                return acc
            return lax.fori_loop(0, _NCHUNK // _UNROLL, chunk,
                                 jnp.zeros((_L,), jnp.int32))

        def splat(v):
            return jnp.full((_L,), v, jnp.int32)

        kk = splat(_K)

        def val_step(_, carry):
            lo, hi = carry
            mid = (lo >> 1) + (hi >> 1) + (lo & hi & 1) + ((lo ^ hi) & 1)
            ge = count_where(lambda k, g: k >= mid) >= kk
            return jnp.where(ge, mid, lo), jnp.where(ge, hi, mid - 1)

        tv, _ = lax.fori_loop(
            0, 32, val_step,
            (splat(-(2 ** 31)), splat(2 ** 31 - 1)))

        c_gt = count_where(lambda k, g: k > tv)
        r = kk - c_gt

        def idx_step(_, carry):
            lo, hi = carry
            mid = (lo + hi) >> 1
            ge = count_where(lambda k, g: (k == tv) & (g <= mid)) >= r
            return jnp.where(ge, lo, mid + 1), jnp.where(ge, mid, hi)

        _, j = lax.fori_loop(0, 13, idx_step,
                             (splat(0), splat(_S - 1)))

        def emit(i, carry):
            for u in range(_UNROLL):
                c = i * _UNROLL + u
                sl = pl.ds(c * _L, _L)
                k = keys_v[sl]
                gidx = c * _L + lax.iota(jnp.int32, _L)
                mask = (k > tv) | ((k == tv) & (gidx <= j))
                x = lg_v[sl]
                score = 1.0 / (1.0 + jnp.exp(-x))
                w_v[sl] = jnp.where(mask, score, jnp.float32(0.0))
                m_v[sl] = mask.astype(jnp.int32)
            return carry

        lax.fori_loop(0, _NCHUNK // _UNROLL, emit, jnp.int32(0))
        pltpu.sync_copy(w_v, w_hbm.at[wid])
        pltpu.sync_copy(m_v, m_hbm.at[wid])


def _sc_hist_body(lg_hbm, w_hbm, m_hbm, lg_v, keys_v, hist_v, ck_v, ci_v,
                  w_v, m_v):
    wid = lax.axis_index("s") * 2 + lax.axis_index("c")

    @pl.when(wid < _B)
    def _():
        pltpu.sync_copy(lg_hbm.at[wid], lg_v)

        zero16 = jnp.zeros((_L,), jnp.int32)
        one16 = jnp.full((_L,), 1, jnp.int32)
        lane = lax.iota(jnp.int32, _L)

        def clr(i, carry):
            for u in range(_UNROLL):
                hist_v[pl.ds((i * _UNROLL + u) * _L, _L)] = zero16
            return carry

        lax.fori_loop(0, 256 // _UNROLL, clr, jnp.int32(0))

        def mk(i, carry):
            for u in range(_UNROLL):
                sl = pl.ds((i * _UNROLL + u) * _L, _L)
                v = lg_v[sl]
                v = jnp.where(v == 0.0, jnp.float32(0.0), v)
                b = lax.bitcast_convert_type(v, jnp.int32)
                k = jnp.where(b >= 0, b, b ^ jnp.int32(0x7FFFFFFF))
                keys_v[sl] = k
                addr = (((k >> 24) + 128) << 4) + lane
                plsc.addupdate_scatter(hist_v, [addr], one16)
            return carry

        lax.fori_loop(0, _NCHUNK // _UNROLL, mk, jnp.int32(0))

        def bscan(i, carry):
            csum, tb, c_above, nc = carry
            b = 255 - i
            h = hist_v[pl.ds(b * _L, _L)]
            s = jnp.sum(h)
            upd = (tb < 0) & (csum + s >= _K)
            tb = jnp.where(upd, b, tb)
            c_above = jnp.where(upd, csum, c_above)
            nc = jnp.where(upd, s, nc)
            csum = csum + jnp.where(tb < 0, s, 0)
            return csum, tb, c_above, nc

        _, tb, c_above, nc = lax.fori_loop(
            0, 256, bscan,
            (jnp.int32(0), jnp.int32(-1), jnp.int32(0), jnp.int32(0)))

        def compact(i, off):
            for u in range(_UNROLL):
                c = i * _UNROLL + u
                sl = pl.ds(c * _L, _L)
                k = keys_v[sl]
                mm = ((k >> 24) + 128) == tb
                plsc.store_compressed(ck_v.at[pl.ds(off, _L)], k, mask=mm)
                plsc.store_compressed(ci_v.at[pl.ds(off, _L)], c * _L + lane,
                                      mask=mm)
                off = off + jnp.sum(mm.astype(jnp.int32))
            return off

        lax.fori_loop(0, _NCHUNK // _UNROLL, compact, jnp.int32(0))

        ncc = (nc + _L - 1) >> 4

        def count_cand(pred):
            def chunk(i, acc):
                sl = pl.ds(i * _L, _L)
                k = ck_v[sl]
                gi = ci_v[sl]
                valid = (i * _L + lane) < nc
                return acc + plsc.all_reduce_population_count(
                    pred(k, gi) & valid)
            return lax.fori_loop(0, ncc, chunk, zero16)

        def splat(v):
            return jnp.full((_L,), v, jnp.int32)

        r1 = splat(_K - c_above)

        def val_step(_, carry):
            lo, hi = carry
            mid = (lo >> 1) + (hi >> 1) + (lo & hi & 1) + ((lo ^ hi) & 1)
            ge = count_cand(lambda k, g: k >= mid) >= r1
            return jnp.where(ge, mid, lo), jnp.where(ge, hi, mid - 1)

        tv, _ = lax.fori_loop(
            0, 32, val_step,
            (splat(-(2 ** 31)), splat(2 ** 31 - 1)))

        c_gt = splat(c_above) + count_cand(lambda k, g: k > tv)
        r = splat(_K) - c_gt

        def idx_step(_, carry):
            lo, hi = carry
            mid = (lo + hi) >> 1
            ge = count_cand(lambda k, g: (k == tv) & (g <= mid)) >= r
            return jnp.where(ge, lo, mid + 1), jnp.where(ge, mid, hi)

        _, j = lax.fori_loop(0, 13, idx_step, (splat(0), splat(_S - 1)))

        def emit(i, carry):
            for u in range(_UNROLL):
                c = i * _UNROLL + u
                sl = pl.ds(c * _L, _L)
                k = keys_v[sl]
                gidx = c * _L + lane
                mask = (k > tv) | ((k == tv) & (gidx <= j))
                x = lg_v[sl]
                score = 1.0 / (1.0 + jnp.exp(-x))
                w_v[sl] = jnp.where(mask, score, jnp.float32(0.0))
                m_v[sl] = mask.astype(jnp.int32)
            return carry

        lax.fori_loop(0, _NCHUNK // _UNROLL, emit, jnp.int32(0))
        pltpu.sync_copy(w_v, w_hbm.at[wid])
        pltpu.sync_copy(m_v, m_hbm.at[wid])


def _sc_select_hist(logits3):
    logits2 = logits3.reshape(_B, _S)
    sel = pl.kernel(
        _sc_hist_body,
        out_type=[
            jax.ShapeDtypeStruct((_B, _S), jnp.float32),
            jax.ShapeDtypeStruct((_B, _S), jnp.int32),
        ],
        mesh=plsc.VectorSubcoreMesh(core_axis_name="c", subcore_axis_name="s"),
        compiler_params=pltpu.CompilerParams(needs_layout_passes=False),
        scratch_types=[
            pltpu.VMEM((_S,), jnp.float32),
            pltpu.VMEM((_S,), jnp.int32),
            pltpu.VMEM((256 * _L,), jnp.int32),
            pltpu.VMEM((_S + _L,), jnp.int32),
            pltpu.VMEM((_S + _L,), jnp.int32),
            pltpu.VMEM((_S,), jnp.float32),
            pltpu.VMEM((_S,), jnp.int32),
        ],
    )
    return sel(logits2)


def _sc_select(logits3):
    logits2 = logits3.reshape(_B, _S)
    sel = pl.kernel(
        _sc_select_body,
        out_type=[
            jax.ShapeDtypeStruct((_B, _S), jnp.float32),
            jax.ShapeDtypeStruct((_B, _S), jnp.int32),
        ],
        mesh=plsc.VectorSubcoreMesh(core_axis_name="c", subcore_axis_name="s"),
        compiler_params=pltpu.CompilerParams(needs_layout_passes=False),
        scratch_types=[
            pltpu.VMEM((_S,), jnp.float32),
            pltpu.VMEM((_S,), jnp.int32),
            pltpu.VMEM((_S,), jnp.float32),
            pltpu.VMEM((_S,), jnp.int32),
        ],
    )
    return sel(logits2)


def kernel(hidden_states, W1, b1, W2, b2):
    b1r = b1.reshape(1, _H4)
    w2t = jnp.zeros((8, _H4), jnp.float32).at[0].set(W2[:, 0])
    b2r = b2.reshape(1, 1)

    logits3 = pl.pallas_call(
        _logits_body,
        grid=(_B, _NS),
        in_specs=[
            pl.BlockSpec((1, _TILE_S, _HIDDEN), lambda b, s: (b, s, 0)),
            pl.BlockSpec((_HIDDEN, _H4), lambda b, s: (0, 0)),
            pl.BlockSpec((1, _H4), lambda b, s: (0, 0)),
            pl.BlockSpec((8, _H4), lambda b, s: (0, 0)),
            pl.BlockSpec((1, 1), lambda b, s: (0, 0)),
        ],
        out_specs=pl.BlockSpec((1, 1, _TILE_S), lambda b, s: (b, 0, s)),
        out_shape=jax.ShapeDtypeStruct((_B, 1, _S), jnp.float32),
    )(hidden_states, W1, b1r, w2t, b2r)

    if _USE_SC_SELECT == "hist":
        weights, mask = _sc_select_hist(logits3)
    elif _USE_SC_SELECT == "bisect":
        weights, mask = _sc_select(logits3)
    else:
        weights, mask = _tc_select(logits3)
    return (weights.reshape(_B, _S), mask.reshape(_B, _S).astype(bool))


# speedup vs baseline: 2.3500x; 1.0036x over previous
import functools

import jax
import jax.numpy as jnp
from jax import lax
from jax.experimental import pallas as pl
from jax.experimental.pallas import tpu as pltpu
from jax.experimental.pallas import tpu_sc as plsc

_B = 4
_S = 4096
_HIDDEN = 2048
_H4 = _HIDDEN // 4
_K = max(1, min(int(0.5 * _S), _S))
_TILE_S = 2048
_NS = _S // _TILE_S
_L = 16
_NCHUNK = _S // _L
_UNROLL = 8

_USE_SC_SELECT = "hist"


def _logits_body(x_ref, w1_ref, b1_ref, w2t_ref, b2_ref, lg_ref):
    x = x_ref[0]
    h = jnp.dot(x, w1_ref[...], preferred_element_type=jnp.float32)
    h = h + b1_ref[...]
    h = 0.5 * h * (1.0 + lax.erf(h * 0.7071067811865476))
    lg8 = lax.dot_general(w2t_ref[...], h, (((1,), (1,)), ((), ())),
                          preferred_element_type=jnp.float32)
    lg_ref[...] = (lg8[0:1] + b2_ref[...])[None]


def _tc_select_body(lg_ref, w_ref, m_ref):
    logits = lg_ref[...]
    zc = jnp.where(logits == 0.0, jnp.float32(0.0), logits)
    bits = lax.bitcast_convert_type(zc, jnp.int32)
    keys = jnp.where(bits >= 0, bits, bits ^ jnp.int32(0x7FFFFFFF))
    idx = lax.broadcasted_iota(jnp.int32, (1, 1, _S), 2)

    def count_ge(t):
        return jnp.sum((keys >= t).astype(jnp.int32))

    def val_step(_, carry):
        lo, hi = carry
        mid = (lo >> 1) + (hi >> 1) + (lo & hi & 1) + ((lo ^ hi) & 1)
        ge = count_ge(mid) >= _K
        return jnp.where(ge, mid, lo), jnp.where(ge, hi, mid - 1)

    tv, _ = lax.fori_loop(
        0, 32, val_step,
        (jnp.int32(-(2 ** 31)), jnp.int32(2 ** 31 - 1)))

    eq = keys == tv
    c_gt = jnp.sum((keys > tv).astype(jnp.int32))
    r = _K - c_gt

    def idx_step(_, carry):
        lo, hi = carry
        mid = (lo + hi) >> 1
        ge = jnp.sum((eq & (idx <= mid)).astype(jnp.int32)) >= r
        return jnp.where(ge, lo, mid + 1), jnp.where(ge, mid, hi)

    _, j = lax.fori_loop(0, 13, idx_step, (jnp.int32(0), jnp.int32(_S - 1)))

    mask = (keys > tv) | (eq & (idx <= j))
    scores = jax.nn.sigmoid(logits)
    w_ref[...] = jnp.where(mask, scores, jnp.float32(0.0))
    m_ref[...] = mask.astype(jnp.int32)


def _tc_select(logits3):
    weights, mask = pl.pallas_call(
        _tc_select_body,
        grid=(_B,),
        in_specs=[pl.BlockSpec((1, 1, _S), lambda b: (b, 0, 0))],
        out_specs=[
            pl.BlockSpec((1, 1, _S), lambda b: (b, 0, 0)),
            pl.BlockSpec((1, 1, _S), lambda b: (b, 0, 0)),
        ],
        out_shape=[
            jax.ShapeDtypeStruct((_B, 1, _S), jnp.float32),
            jax.ShapeDtypeStruct((_B, 1, _S), jnp.int32),
        ],
    )(logits3)
    return weights.reshape(_B, _S), mask.reshape(_B, _S)


def _sc_select_body(lg_hbm, w_hbm, m_hbm, lg_v, keys_v, w_v, m_v):
    wid = lax.axis_index("s") * 2 + lax.axis_index("c")

    @pl.when(wid < _B)
    def _():
        pltpu.sync_copy(lg_hbm.at[wid], lg_v)

        def mk(i, carry):
            for u in range(_UNROLL):
                sl = pl.ds((i * _UNROLL + u) * _L, _L)
                v = lg_v[sl]
                v = jnp.where(v == 0.0, jnp.float32(0.0), v)
                b = lax.bitcast_convert_type(v, jnp.int32)
                keys_v[sl] = jnp.where(b >= 0, b, b ^ jnp.int32(0x7FFFFFFF))
            return carry

        lax.fori_loop(0, _NCHUNK // _UNROLL, mk, jnp.int32(0))

        def count_where(pred):
            def chunk(i, acc):
                for u in range(_UNROLL):
                    c = i * _UNROLL + u
                    k = keys_v[pl.ds(c * _L, _L)]
                    gidx = c * _L + lax.iota(jnp.int32, _L)
                    acc = acc + plsc.all_reduce_population_count(pred(k, gidx))
                return acc
            return lax.fori_loop(0, _NCHUNK // _UNROLL, chunk,
                                 jnp.zeros((_L,), jnp.int32))

        def splat(v):
            return jnp.full((_L,), v, jnp.int32)

        kk = splat(_K)

        def val_step(_, carry):
            lo, hi = carry
            mid = (lo >> 1) + (hi >> 1) + (lo & hi & 1) + ((lo ^ hi) & 1)
            ge = count_where(lambda k, g: k >= mid) >= kk
            return jnp.where(ge, mid, lo), jnp.where(ge, hi, mid - 1)

        tv, _ = lax.fori_loop(
            0, 32, val_step,
            (splat(-(2 ** 31)), splat(2 ** 31 - 1)))

        c_gt = count_where(lambda k, g: k > tv)
        r = kk - c_gt

        def idx_step(_, carry):
            lo, hi = carry
            mid = (lo + hi) >> 1
            ge = count_where(lambda k, g: (k == tv) & (g <= mid)) >= r
            return jnp.where(ge, lo, mid + 1), jnp.where(ge, mid, hi)

        _, j = lax.fori_loop(0, 13, idx_step,
                             (splat(0), splat(_S - 1)))

        def emit(i, carry):
            for u in range(_UNROLL):
                c = i * _UNROLL + u
                sl = pl.ds(c * _L, _L)
                k = keys_v[sl]
                gidx = c * _L + lax.iota(jnp.int32, _L)
                mask = (k > tv) | ((k == tv) & (gidx <= j))
                x = lg_v[sl]
                score = 1.0 / (1.0 + jnp.exp(-x))
                w_v[sl] = jnp.where(mask, score, jnp.float32(0.0))
                m_v[sl] = mask.astype(jnp.int32)
            return carry

        lax.fori_loop(0, _NCHUNK // _UNROLL, emit, jnp.int32(0))
        pltpu.sync_copy(w_v, w_hbm.at[wid])
        pltpu.sync_copy(m_v, m_hbm.at[wid])


def _sc_hist_body(lg_hbm, w_hbm, m_hbm, lg_v, keys_v, hist_v, ck_v, ci_v,
                  w_v, m_v):
    wid = lax.axis_index("s") * 2 + lax.axis_index("c")

    @pl.when(wid < _B)
    def _():
        pltpu.sync_copy(lg_hbm.at[wid], lg_v)

        zero16 = jnp.zeros((_L,), jnp.int32)
        one16 = jnp.full((_L,), 1, jnp.int32)
        lane = lax.iota(jnp.int32, _L)

        def clr(i, carry):
            for u in range(_UNROLL):
                hist_v[pl.ds((i * _UNROLL + u) * _L, _L)] = zero16
            return carry

        lax.fori_loop(0, 256 // _UNROLL, clr, jnp.int32(0))

        def mk(i, carry):
            for u in range(_UNROLL):
                sl = pl.ds((i * _UNROLL + u) * _L, _L)
                v = lg_v[sl]
                v = jnp.where(v == 0.0, jnp.float32(0.0), v)
                b = lax.bitcast_convert_type(v, jnp.int32)
                k = jnp.where(b >= 0, b, b ^ jnp.int32(0x7FFFFFFF))
                keys_v[sl] = k
                addr = (((k >> 24) + 128) << 4) + lane
                plsc.addupdate_scatter(hist_v, [addr], one16)
            return carry

        lax.fori_loop(0, _NCHUNK // _UNROLL, mk, jnp.int32(0))

        def bscan(i, carry):
            sums = []
            for u in range(_UNROLL):
                b = 255 - (i * _UNROLL + u)
                sums.append(jnp.sum(hist_v[pl.ds(b * _L, _L)]))
            for u in range(_UNROLL):
                csum, tb, c_above, nc = carry
                b = 255 - (i * _UNROLL + u)
                s = sums[u]
                upd = (tb < 0) & (csum + s >= _K)
                tb = jnp.where(upd, b, tb)
                c_above = jnp.where(upd, csum, c_above)
                nc = jnp.where(upd, s, nc)
                csum = csum + jnp.where(tb < 0, s, 0)
                carry = (csum, tb, c_above, nc)
            return carry

        _, tb, c_above, nc = lax.fori_loop(
            0, 256 // _UNROLL, bscan,
            (jnp.int32(0), jnp.int32(-1), jnp.int32(0), jnp.int32(0)))

        def compact(i, off):
            for u in range(_UNROLL):
                c = i * _UNROLL + u
                sl = pl.ds(c * _L, _L)
                k = keys_v[sl]
                mm = ((k >> 24) + 128) == tb
                plsc.store_compressed(ck_v.at[pl.ds(off, _L)], k, mask=mm)
                plsc.store_compressed(ci_v.at[pl.ds(off, _L)], c * _L + lane,
                                      mask=mm)
                off = off + plsc.all_reduce_population_count(mm)[0]
            return off

        lax.fori_loop(0, _NCHUNK // _UNROLL, compact, jnp.int32(0))

        ncc = (nc + _L - 1) >> 4

        def count_cand(pred):
            def chunk(i, acc):
                sl = pl.ds(i * _L, _L)
                k = ck_v[sl]
                gi = ci_v[sl]
                valid = (i * _L + lane) < nc
                return acc + plsc.all_reduce_population_count(
                    pred(k, gi) & valid)
            return lax.fori_loop(0, ncc, chunk, zero16)

        def splat(v):
            return jnp.full((_L,), v, jnp.int32)

        r1 = splat(_K - c_above)

        def val_step(_, carry):
            lo, hi = carry
            mid = (lo >> 1) + (hi >> 1) + (lo & hi & 1) + ((lo ^ hi) & 1)
            ge = count_cand(lambda k, g: k >= mid) >= r1
            return jnp.where(ge, mid, lo), jnp.where(ge, hi, mid - 1)

        tv, _ = lax.fori_loop(
            0, 32, val_step,
            (splat(-(2 ** 31)), splat(2 ** 31 - 1)))

        c_gt = splat(c_above) + count_cand(lambda k, g: k > tv)
        r = splat(_K) - c_gt

        def idx_step(_, carry):
            lo, hi = carry
            mid = (lo + hi) >> 1
            ge = count_cand(lambda k, g: (k == tv) & (g <= mid)) >= r
            return jnp.where(ge, lo, mid + 1), jnp.where(ge, mid, hi)

        _, j = lax.fori_loop(0, 13, idx_step, (splat(0), splat(_S - 1)))

        def emit(i, carry):
            for u in range(_UNROLL):
                c = i * _UNROLL + u
                sl = pl.ds(c * _L, _L)
                k = keys_v[sl]
                gidx = c * _L + lane
                mask = (k > tv) | ((k == tv) & (gidx <= j))
                x = lg_v[sl]
                score = 1.0 / (1.0 + jnp.exp(-x))
                w_v[sl] = jnp.where(mask, score, jnp.float32(0.0))
                m_v[sl] = mask.astype(jnp.int32)
            return carry

        lax.fori_loop(0, _NCHUNK // _UNROLL, emit, jnp.int32(0))
        pltpu.sync_copy(w_v, w_hbm.at[wid])
        pltpu.sync_copy(m_v, m_hbm.at[wid])


def _sc_select_hist(logits3):
    logits2 = logits3.reshape(_B, _S)
    sel = pl.kernel(
        _sc_hist_body,
        out_type=[
            jax.ShapeDtypeStruct((_B, _S), jnp.float32),
            jax.ShapeDtypeStruct((_B, _S), jnp.int32),
        ],
        mesh=plsc.VectorSubcoreMesh(core_axis_name="c", subcore_axis_name="s"),
        compiler_params=pltpu.CompilerParams(needs_layout_passes=False),
        scratch_types=[
            pltpu.VMEM((_S,), jnp.float32),
            pltpu.VMEM((_S,), jnp.int32),
            pltpu.VMEM((256 * _L,), jnp.int32),
            pltpu.VMEM((_S + _L,), jnp.int32),
            pltpu.VMEM((_S + _L,), jnp.int32),
            pltpu.VMEM((_S,), jnp.float32),
            pltpu.VMEM((_S,), jnp.int32),
        ],
    )
    return sel(logits2)


def _sc_select(logits3):
    logits2 = logits3.reshape(_B, _S)
    sel = pl.kernel(
        _sc_select_body,
        out_type=[
            jax.ShapeDtypeStruct((_B, _S), jnp.float32),
            jax.ShapeDtypeStruct((_B, _S), jnp.int32),
        ],
        mesh=plsc.VectorSubcoreMesh(core_axis_name="c", subcore_axis_name="s"),
        compiler_params=pltpu.CompilerParams(needs_layout_passes=False),
        scratch_types=[
            pltpu.VMEM((_S,), jnp.float32),
            pltpu.VMEM((_S,), jnp.int32),
            pltpu.VMEM((_S,), jnp.float32),
            pltpu.VMEM((_S,), jnp.int32),
        ],
    )
    return sel(logits2)


def kernel(hidden_states, W1, b1, W2, b2):
    b1r = b1.reshape(1, _H4)
    w2t = jnp.zeros((8, _H4), jnp.float32).at[0].set(W2[:, 0])
    b2r = b2.reshape(1, 1)

    logits3 = pl.pallas_call(
        _logits_body,
        grid=(_B, _NS),
        in_specs=[
            pl.BlockSpec((1, _TILE_S, _HIDDEN), lambda b, s: (b, s, 0)),
            pl.BlockSpec((_HIDDEN, _H4), lambda b, s: (0, 0)),
            pl.BlockSpec((1, _H4), lambda b, s: (0, 0)),
            pl.BlockSpec((8, _H4), lambda b, s: (0, 0)),
            pl.BlockSpec((1, 1), lambda b, s: (0, 0)),
        ],
        out_specs=pl.BlockSpec((1, 1, _TILE_S), lambda b, s: (b, 0, s)),
        out_shape=jax.ShapeDtypeStruct((_B, 1, _S), jnp.float32),
    )(hidden_states, W1, b1r, w2t, b2r)

    if _USE_SC_SELECT == "hist":
        weights, mask = _sc_select_hist(logits3)
    elif _USE_SC_SELECT == "bisect":
        weights, mask = _sc_select(logits3)
    else:
        weights, mask = _tc_select(logits3)
    return (weights.reshape(_B, _S), mask.reshape(_B, _S).astype(bool))


# speedup vs baseline: 2.6187x; 1.1143x over previous
import functools

import jax
import jax.numpy as jnp
from jax import lax
from jax.experimental import pallas as pl
from jax.experimental.pallas import tpu as pltpu
from jax.experimental.pallas import tpu_sc as plsc

_B = 4
_S = 4096
_HIDDEN = 2048
_H4 = _HIDDEN // 4
_K = max(1, min(int(0.5 * _S), _S))
_TILE_S = 2048
_NS = _S // _TILE_S
_L = 16
_NCHUNK = _S // _L
_UNROLL = 8

_USE_SC_SELECT = "tc"


def _logits_body(x_ref, w1_ref, b1_ref, w2t_ref, b2_ref, lg_ref):
    x = x_ref[0]
    h = jnp.dot(x, w1_ref[...], preferred_element_type=jnp.float32)
    h = h + b1_ref[...]
    h = 0.5 * h * (1.0 + lax.erf(h * 0.7071067811865476))
    lg8 = lax.dot_general(w2t_ref[...], h, (((1,), (1,)), ((), ())),
                          preferred_element_type=jnp.float32)
    lg_ref[...] = (lg8[0:1] + b2_ref[...])[None]


def _tc_select_body(lg_ref, w_ref, m_ref):
    logits = lg_ref[...]
    zc = jnp.where(logits == 0.0, jnp.float32(0.0), logits)
    bits = lax.bitcast_convert_type(zc, jnp.int32)
    keys = jnp.where(bits >= 0, bits, bits ^ jnp.int32(0x7FFFFFFF))
    idx = lax.broadcasted_iota(jnp.int32, (1, 1, _S), 2)

    def count_ge(t):
        return jnp.sum((keys >= t).astype(jnp.int32))

    def val_step(_, carry):
        lo, hi = carry
        mid = (lo >> 1) + (hi >> 1) + (lo & hi & 1) + ((lo ^ hi) & 1)
        ge = count_ge(mid) >= _K
        return jnp.where(ge, mid, lo), jnp.where(ge, hi, mid - 1)

    tv, _ = lax.fori_loop(
        0, 32, val_step,
        (jnp.int32(-(2 ** 31)), jnp.int32(2 ** 31 - 1)))

    eq = keys == tv
    c_gt = jnp.sum((keys > tv).astype(jnp.int32))
    r = _K - c_gt

    def idx_step(_, carry):
        lo, hi = carry
        mid = (lo + hi) >> 1
        ge = jnp.sum((eq & (idx <= mid)).astype(jnp.int32)) >= r
        return jnp.where(ge, lo, mid + 1), jnp.where(ge, mid, hi)

    _, j = lax.fori_loop(0, 13, idx_step, (jnp.int32(0), jnp.int32(_S - 1)))

    mask = (keys > tv) | (eq & (idx <= j))
    scores = jax.nn.sigmoid(logits)
    w_ref[...] = jnp.where(mask, scores, jnp.float32(0.0))
    m_ref[...] = mask.astype(jnp.int32)


def _tc_select(logits3):
    weights, mask = pl.pallas_call(
        _tc_select_body,
        grid=(_B,),
        in_specs=[pl.BlockSpec((1, 1, _S), lambda b: (b, 0, 0))],
        out_specs=[
            pl.BlockSpec((1, 1, _S), lambda b: (b, 0, 0)),
            pl.BlockSpec((1, 1, _S), lambda b: (b, 0, 0)),
        ],
        out_shape=[
            jax.ShapeDtypeStruct((_B, 1, _S), jnp.float32),
            jax.ShapeDtypeStruct((_B, 1, _S), jnp.int32),
        ],
    )(logits3)
    return weights.reshape(_B, _S), mask.reshape(_B, _S)


def _sc_select_body(lg_hbm, w_hbm, m_hbm, lg_v, keys_v, w_v, m_v):
    wid = lax.axis_index("s") * 2 + lax.axis_index("c")

    @pl.when(wid < _B)
    def _():
        pltpu.sync_copy(lg_hbm.at[wid], lg_v)

        def mk(i, carry):
            for u in range(_UNROLL):
                sl = pl.ds((i * _UNROLL + u) * _L, _L)
                v = lg_v[sl]
                v = jnp.where(v == 0.0, jnp.float32(0.0), v)
                b = lax.bitcast_convert_type(v, jnp.int32)
                keys_v[sl] = jnp.where(b >= 0, b, b ^ jnp.int32(0x7FFFFFFF))
            return carry

        lax.fori_loop(0, _NCHUNK // _UNROLL, mk, jnp.int32(0))

        def count_where(pred):
            def chunk(i, acc):
                for u in range(_UNROLL):
                    c = i * _UNROLL + u
                    k = keys_v[pl.ds(c * _L, _L)]
                    gidx = c * _L + lax.iota(jnp.int32, _L)
                    acc = acc + plsc.all_reduce_population_count(pred(k, gidx))
                return acc
            return lax.fori_loop(0, _NCHUNK // _UNROLL, chunk,
                                 jnp.zeros((_L,), jnp.int32))

        def splat(v):
            return jnp.full((_L,), v, jnp.int32)

        kk = splat(_K)

        def val_step(_, carry):
            lo, hi = carry
            mid = (lo >> 1) + (hi >> 1) + (lo & hi & 1) + ((lo ^ hi) & 1)
            ge = count_where(lambda k, g: k >= mid) >= kk
            return jnp.where(ge, mid, lo), jnp.where(ge, hi, mid - 1)

        tv, _ = lax.fori_loop(
            0, 32, val_step,
            (splat(-(2 ** 31)), splat(2 ** 31 - 1)))

        c_gt = count_where(lambda k, g: k > tv)
        r = kk - c_gt

        def idx_step(_, carry):
            lo, hi = carry
            mid = (lo + hi) >> 1
            ge = count_where(lambda k, g: (k == tv) & (g <= mid)) >= r
            return jnp.where(ge, lo, mid + 1), jnp.where(ge, mid, hi)

        _, j = lax.fori_loop(0, 13, idx_step,
                             (splat(0), splat(_S - 1)))

        def emit(i, carry):
            for u in range(_UNROLL):
                c = i * _UNROLL + u
                sl = pl.ds(c * _L, _L)
                k = keys_v[sl]
                gidx = c * _L + lax.iota(jnp.int32, _L)
                mask = (k > tv) | ((k == tv) & (gidx <= j))
                x = lg_v[sl]
                score = 1.0 / (1.0 + jnp.exp(-x))
                w_v[sl] = jnp.where(mask, score, jnp.float32(0.0))
                m_v[sl] = mask.astype(jnp.int32)
            return carry

        lax.fori_loop(0, _NCHUNK // _UNROLL, emit, jnp.int32(0))
        pltpu.sync_copy(w_v, w_hbm.at[wid])
        pltpu.sync_copy(m_v, m_hbm.at[wid])


def _sc_hist_body(lg_hbm, w_hbm, m_hbm, lg_v, keys_v, hist_v, ck_v, ci_v,
                  w_v, m_v):
    wid = lax.axis_index("s") * 2 + lax.axis_index("c")

    @pl.when(wid < _B)
    def _():
        pltpu.sync_copy(lg_hbm.at[wid], lg_v)

        zero16 = jnp.zeros((_L,), jnp.int32)
        one16 = jnp.full((_L,), 1, jnp.int32)
        lane = lax.iota(jnp.int32, _L)

        def clr(i, carry):
            for u in range(_UNROLL):
                hist_v[pl.ds((i * _UNROLL + u) * _L, _L)] = zero16
            return carry

        lax.fori_loop(0, 256 // _UNROLL, clr, jnp.int32(0))

        def mk(i, carry):
            for u in range(_UNROLL):
                sl = pl.ds((i * _UNROLL + u) * _L, _L)
                v = lg_v[sl]
                v = jnp.where(v == 0.0, jnp.float32(0.0), v)
                b = lax.bitcast_convert_type(v, jnp.int32)
                k = jnp.where(b >= 0, b, b ^ jnp.int32(0x7FFFFFFF))
                keys_v[sl] = k
                addr = (((k >> 24) + 128) << 4) + lane
                plsc.addupdate_scatter(hist_v, [addr], one16)
            return carry

        lax.fori_loop(0, _NCHUNK // _UNROLL, mk, jnp.int32(0))

        def bscan(i, carry):
            sums = []
            for u in range(_UNROLL):
                b = 255 - (i * _UNROLL + u)
                sums.append(jnp.sum(hist_v[pl.ds(b * _L, _L)]))
            for u in range(_UNROLL):
                csum, tb, c_above, nc = carry
                b = 255 - (i * _UNROLL + u)
                s = sums[u]
                upd = (tb < 0) & (csum + s >= _K)
                tb = jnp.where(upd, b, tb)
                c_above = jnp.where(upd, csum, c_above)
                nc = jnp.where(upd, s, nc)
                csum = csum + jnp.where(tb < 0, s, 0)
                carry = (csum, tb, c_above, nc)
            return carry

        _, tb, c_above, nc = lax.fori_loop(
            0, 256 // _UNROLL, bscan,
            (jnp.int32(0), jnp.int32(-1), jnp.int32(0), jnp.int32(0)))

        def compact(i, off):
            for u in range(_UNROLL):
                c = i * _UNROLL + u
                sl = pl.ds(c * _L, _L)
                k = keys_v[sl]
                mm = ((k >> 24) + 128) == tb
                plsc.store_compressed(ck_v.at[pl.ds(off, _L)], k, mask=mm)
                plsc.store_compressed(ci_v.at[pl.ds(off, _L)], c * _L + lane,
                                      mask=mm)
                off = off + plsc.all_reduce_population_count(mm)[0]
            return off

        lax.fori_loop(0, _NCHUNK // _UNROLL, compact, jnp.int32(0))

        ncc = (nc + _L - 1) >> 4

        def count_cand(pred):
            def chunk(i, acc):
                sl = pl.ds(i * _L, _L)
                k = ck_v[sl]
                gi = ci_v[sl]
                valid = (i * _L + lane) < nc
                return acc + plsc.all_reduce_population_count(
                    pred(k, gi) & valid)
            return lax.fori_loop(0, ncc, chunk, zero16)

        def splat(v):
            return jnp.full((_L,), v, jnp.int32)

        r1 = splat(_K - c_above)

        def val_step(_, carry):
            lo, hi = carry
            mid = (lo >> 1) + (hi >> 1) + (lo & hi & 1) + ((lo ^ hi) & 1)
            ge = count_cand(lambda k, g: k >= mid) >= r1
            return jnp.where(ge, mid, lo), jnp.where(ge, hi, mid - 1)

        tv, _ = lax.fori_loop(
            0, 32, val_step,
            (splat(-(2 ** 31)), splat(2 ** 31 - 1)))

        c_gt = splat(c_above) + count_cand(lambda k, g: k > tv)
        r = splat(_K) - c_gt

        def idx_step(_, carry):
            lo, hi = carry
            mid = (lo + hi) >> 1
            ge = count_cand(lambda k, g: (k == tv) & (g <= mid)) >= r
            return jnp.where(ge, lo, mid + 1), jnp.where(ge, mid, hi)

        _, j = lax.fori_loop(0, 13, idx_step, (splat(0), splat(_S - 1)))

        def emit(i, carry):
            for u in range(_UNROLL):
                c = i * _UNROLL + u
                sl = pl.ds(c * _L, _L)
                k = keys_v[sl]
                gidx = c * _L + lane
                mask = (k > tv) | ((k == tv) & (gidx <= j))
                x = lg_v[sl]
                score = 1.0 / (1.0 + jnp.exp(-x))
                w_v[sl] = jnp.where(mask, score, jnp.float32(0.0))
                m_v[sl] = mask.astype(jnp.int32)
            return carry

        lax.fori_loop(0, _NCHUNK // _UNROLL, emit, jnp.int32(0))
        pltpu.sync_copy(w_v, w_hbm.at[wid])
        pltpu.sync_copy(m_v, m_hbm.at[wid])


def _sc_select_hist(logits3):
    logits2 = logits3.reshape(_B, _S)
    sel = pl.kernel(
        _sc_hist_body,
        out_type=[
            jax.ShapeDtypeStruct((_B, _S), jnp.float32),
            jax.ShapeDtypeStruct((_B, _S), jnp.int32),
        ],
        mesh=plsc.VectorSubcoreMesh(core_axis_name="c", subcore_axis_name="s"),
        compiler_params=pltpu.CompilerParams(needs_layout_passes=False),
        scratch_types=[
            pltpu.VMEM((_S,), jnp.float32),
            pltpu.VMEM((_S,), jnp.int32),
            pltpu.VMEM((256 * _L,), jnp.int32),
            pltpu.VMEM((_S + _L,), jnp.int32),
            pltpu.VMEM((_S + _L,), jnp.int32),
            pltpu.VMEM((_S,), jnp.float32),
            pltpu.VMEM((_S,), jnp.int32),
        ],
    )
    return sel(logits2)


def _sc_select(logits3):
    logits2 = logits3.reshape(_B, _S)
    sel = pl.kernel(
        _sc_select_body,
        out_type=[
            jax.ShapeDtypeStruct((_B, _S), jnp.float32),
            jax.ShapeDtypeStruct((_B, _S), jnp.int32),
        ],
        mesh=plsc.VectorSubcoreMesh(core_axis_name="c", subcore_axis_name="s"),
        compiler_params=pltpu.CompilerParams(needs_layout_passes=False),
        scratch_types=[
            pltpu.VMEM((_S,), jnp.float32),
            pltpu.VMEM((_S,), jnp.int32),
            pltpu.VMEM((_S,), jnp.float32),
            pltpu.VMEM((_S,), jnp.int32),
        ],
    )
    return sel(logits2)


def kernel(hidden_states, W1, b1, W2, b2):
    b1r = b1.reshape(1, _H4)
    w2t = jnp.zeros((8, _H4), jnp.float32).at[0].set(W2[:, 0])
    b2r = b2.reshape(1, 1)

    logits3 = pl.pallas_call(
        _logits_body,
        grid=(_B, _NS),
        in_specs=[
            pl.BlockSpec((1, _TILE_S, _HIDDEN), lambda b, s: (b, s, 0)),
            pl.BlockSpec((_HIDDEN, _H4), lambda b, s: (0, 0)),
            pl.BlockSpec((1, _H4), lambda b, s: (0, 0)),
            pl.BlockSpec((8, _H4), lambda b, s: (0, 0)),
            pl.BlockSpec((1, 1), lambda b, s: (0, 0)),
        ],
        out_specs=pl.BlockSpec((1, 1, _TILE_S), lambda b, s: (b, 0, s)),
        out_shape=jax.ShapeDtypeStruct((_B, 1, _S), jnp.float32),
    )(hidden_states, W1, b1r, w2t, b2r)

    if _USE_SC_SELECT == "hist":
        weights, mask = _sc_select_hist(logits3)
    elif _USE_SC_SELECT == "bisect":
        weights, mask = _sc_select(logits3)
    else:
        weights, mask = _tc_select(logits3)
    return (weights.reshape(_B, _S), mask.reshape(_B, _S).astype(bool))
